# baseline (speedup 1.0000x reference)
import functools

import jax
import jax.numpy as jnp
from jax import lax
from jax.experimental import pallas as pl
from jax.experimental.pallas import tpu as pltpu
from jax.experimental.pallas import tpu_sc as plsc

N_E = 8192
E_DIM = 256
NB = 16
M_PER_B = 1024
B_TOT = NB * M_PER_B
N_BLK = 2048
N_STEPS = N_E // N_BLK
LOSS_SCALE = 1.25 / (B_TOT * E_DIM)


W_BOUNDS = (0, 2736, 5472, N_E)


def _z2_body(zf_ref, o_ref):
    zfb = zf_ref[...]
    o_ref[...] = jnp.sum(zfb * zfb, axis=1)


def _z2_rows(zf2d):
    return pl.pallas_call(
        _z2_body,
        grid=(NB,),
        in_specs=[pl.BlockSpec((M_PER_B, E_DIM), lambda b: (b, 0))],
        out_specs=pl.BlockSpec((M_PER_B,), lambda b: (b,)),
        out_shape=jax.ShapeDtypeStruct((B_TOT,), jnp.float32),
    )(zf2d)


def _dist_argmin_body(z_ref, z2_ref, e_ref, idx_ref, loss_ref, rmin_ref,
                      ridx_ref):
    b = pl.program_id(0)
    n = pl.program_id(1)
    n0 = n * N_BLK

    @pl.when(n == 0)
    def _():
        rmin_ref[...] = jnp.full((3, M_PER_B), jnp.inf, jnp.float32)
        ridx_ref[...] = jnp.zeros((3, M_PER_B), jnp.int32)

    zb = z_ref[0]
    eb = e_ref[...]
    mm = lax.dot_general(eb, zb, (((1,), (0,)), ((), ())),
                         preferred_element_type=jnp.float32,
                         precision=lax.Precision.DEFAULT)
    z2 = z2_ref[0, 0]
    e2 = jnp.sum(eb * eb, axis=1)
    d = (z2[None, :] + e2[:, None]) - 2.0 * mm
    rows = lax.broadcasted_iota(jnp.int32, d.shape, 0)
    gidx = rows + n0
    for w in range(3):
        lo, hi = W_BOUNDS[w], W_BOUNDS[w + 1]
        inwin = (gidx >= lo) & (gidx < hi)
        dm = jnp.where(inwin, d, jnp.inf)
        bmin = jnp.min(dm, axis=0)
        bidx = jnp.min(jnp.where(dm == bmin[None, :], gidx, jnp.int32(N_E)),
                       axis=0)
        better = bmin < rmin_ref[w]
        rmin_ref[w] = jnp.where(better, bmin, rmin_ref[w])
        ridx_ref[w] = jnp.where(better, bidx, ridx_ref[w])

    @pl.when(n == N_STEPS - 1)
    def _():
        v0, v1, v2 = rmin_ref[0], rmin_ref[1], rmin_ref[2]
        i0, i1, i2 = ridx_ref[0], ridx_ref[1], ridx_ref[2]
        sel_v, sel_i = v0, i0
        carry = v0.astype(jnp.bfloat16).astype(jnp.float32)
        win1 = v1 < carry
        sel_v = jnp.where(win1, v1, sel_v)
        sel_i = jnp.where(win1, i1, sel_i)
        carry = jnp.where(win1, v1.astype(jnp.bfloat16).astype(jnp.float32),
                          carry)
        win2 = v2 < carry
        sel_v = jnp.where(win2, v2, sel_v)
        sel_i = jnp.where(win2, i2, sel_i)
        idx_ref[...] = sel_i
        s = jnp.sum(sel_v)
        prev = jnp.where(b == 0, jnp.float32(0.0), loss_ref[0, 0])
        tot = prev + s
        loss_ref[0, 0] = jnp.where(b == NB - 1, tot * LOSS_SCALE, tot)


def _dist_argmin(z3, z2rows, embedding, interpret=False):
    return pl.pallas_call(
        _dist_argmin_body,
        grid=(NB, N_STEPS),
        in_specs=[
            pl.BlockSpec((1, E_DIM, M_PER_B), lambda b, n: (b, 0, 0)),
            pl.BlockSpec((1, 1, M_PER_B), lambda b, n: (b, 0, 0)),
            pl.BlockSpec((N_BLK, E_DIM), lambda b, n: (n, 0)),
        ],
        out_specs=[
            pl.BlockSpec((M_PER_B,), lambda b, n: (b,)),
            pl.BlockSpec((1, 1), lambda b, n: (0, 0),
                         memory_space=pltpu.SMEM),
        ],
        out_shape=[
            jax.ShapeDtypeStruct((B_TOT,), jnp.int32),
            jax.ShapeDtypeStruct((1, 1), jnp.float32),
        ],
        scratch_shapes=[
            pltpu.VMEM((3, M_PER_B), jnp.float32),
            pltpu.VMEM((3, M_PER_B), jnp.int32),
        ],
        interpret=interpret,
    )(z3, z2rows, embedding)


def _sc_gather(embedding, idx):
    NC, NS = 2, 16
    NW = NC * NS
    CH = 128
    per_w = B_TOT // NW
    n_ch = per_w // CH
    mesh = plsc.VectorSubcoreMesh(core_axis_name="c", subcore_axis_name="s")

    @functools.partial(
        pl.kernel, mesh=mesh,
        out_type=jax.ShapeDtypeStruct((B_TOT, E_DIM), jnp.float32),
        scratch_types=[
            pltpu.VMEM((CH,), jnp.int32),
            pltpu.VMEM((CH, E_DIM), jnp.float32),
            pltpu.SemaphoreType.DMA,
        ],
    )
    def gk(table_hbm, idx_hbm, out_hbm, idx_v, rows_v, sem):
        wid = lax.axis_index("s") * NC + lax.axis_index("c")
        base = wid * per_w
        for j in range(n_ch):
            off = base + j * CH
            pltpu.sync_copy(idx_hbm.at[pl.ds(off, CH)], idx_v)
            pltpu.async_copy(table_hbm.at[idx_v], rows_v, sem).wait()
            pltpu.sync_copy(rows_v, out_hbm.at[pl.ds(off, CH)])

    return gk(embedding, idx)


def kernel(z, embedding):
    z3 = z.reshape(NB, E_DIM, M_PER_B)
    zf2d = jnp.transpose(z, (0, 2, 3, 1)).reshape(B_TOT, E_DIM)
    z2rows = _z2_rows(zf2d).reshape(NB, 1, M_PER_B)
    idx_flat, loss = _dist_argmin(z3, z2rows, embedding)
    zq_flat = _sc_gather(embedding, idx_flat)
    z_q = zq_flat.reshape(NB, 32, 32, E_DIM).transpose(0, 3, 1, 2)
    return z_q, loss[0, 0], idx_flat


# speedup vs baseline: 1.8575x; 1.8575x over previous
import functools

import jax
import jax.numpy as jnp
from jax import lax
from jax.experimental import pallas as pl
from jax.experimental.pallas import tpu as pltpu
from jax.experimental.pallas import tpu_sc as plsc

N_E = 8192
E_DIM = 256
NB = 16
M_PER_B = 1024
B_TOT = NB * M_PER_B
LOSS_SCALE = 1.25 / (B_TOT * E_DIM)

N_BLK = 2736
N_STEPS = 3
N_PAD = N_BLK * N_STEPS


def _z2_body(zf_ref, o_ref):
    zfb = zf_ref[...]
    o_ref[...] = jnp.sum(zfb * zfb, axis=1)


def _z2_rows(zf2d):
    return pl.pallas_call(
        _z2_body,
        grid=(NB,),
        in_specs=[pl.BlockSpec((M_PER_B, E_DIM), lambda b: (b, 0))],
        out_specs=pl.BlockSpec((M_PER_B,), lambda b: (b,)),
        out_shape=jax.ShapeDtypeStruct((B_TOT,), jnp.float32),
    )(zf2d)


def _dist_argmin_body(z_ref, z2_ref, e_ref, idx_ref, loss_ref, selv_ref,
                      seli_ref, carry_ref):
    b = pl.program_id(0)
    n = pl.program_id(1)
    n0 = n * N_BLK

    zb = z_ref[0]
    eb = e_ref[...]
    mm = lax.dot_general(eb, zb, (((1,), (0,)), ((), ())),
                         preferred_element_type=jnp.float32,
                         precision=lax.Precision.DEFAULT)
    z2 = z2_ref[0, 0]
    e2 = jnp.sum(eb * eb, axis=1)
    d = (z2[None, :] + e2[:, None]) - 2.0 * mm
    gidx = lax.broadcasted_iota(jnp.int32, d.shape, 0) + n0
    vw = jnp.min(d, axis=0)
    iw = jnp.min(jnp.where(d == vw[None, :], gidx, jnp.int32(N_PAD)),
                 axis=0)
    vw_bf = vw.astype(jnp.bfloat16).astype(jnp.float32)

    @pl.when(n == 0)
    def _():
        selv_ref[...] = vw
        seli_ref[...] = iw
        carry_ref[...] = vw_bf

    @pl.when(n > 0)
    def _():
        win = vw < carry_ref[...]
        selv_ref[...] = jnp.where(win, vw, selv_ref[...])
        seli_ref[...] = jnp.where(win, iw, seli_ref[...])
        carry_ref[...] = jnp.where(win, vw_bf, carry_ref[...])

    @pl.when(n == N_STEPS - 1)
    def _():
        idx_ref[...] = seli_ref[...]
        s = jnp.sum(selv_ref[...])
        prev = jnp.where(b == 0, jnp.float32(0.0), loss_ref[0, 0])
        tot = prev + s
        loss_ref[0, 0] = jnp.where(b == NB - 1, tot * LOSS_SCALE, tot)


def _dist_argmin(z3, z2rows, e_pad, interpret=False):
    return pl.pallas_call(
        _dist_argmin_body,
        grid=(NB, N_STEPS),
        in_specs=[
            pl.BlockSpec((1, E_DIM, M_PER_B), lambda b, n: (b, 0, 0)),
            pl.BlockSpec((1, 1, M_PER_B), lambda b, n: (b, 0, 0)),
            pl.BlockSpec((N_BLK, E_DIM), lambda b, n: (n, 0)),
        ],
        out_specs=[
            pl.BlockSpec((M_PER_B,), lambda b, n: (b,)),
            pl.BlockSpec((1, 1), lambda b, n: (0, 0),
                         memory_space=pltpu.SMEM),
        ],
        out_shape=[
            jax.ShapeDtypeStruct((B_TOT,), jnp.int32),
            jax.ShapeDtypeStruct((1, 1), jnp.float32),
        ],
        scratch_shapes=[
            pltpu.VMEM((M_PER_B,), jnp.float32),
            pltpu.VMEM((M_PER_B,), jnp.int32),
            pltpu.VMEM((M_PER_B,), jnp.float32),
        ],
        interpret=interpret,
    )(z3, z2rows, e_pad)


def _sc_gather(embedding, idx):
    NC, NS = 2, 16
    NW = NC * NS
    CH = 128
    per_w = B_TOT // NW
    n_ch = per_w // CH
    mesh = plsc.VectorSubcoreMesh(core_axis_name="c", subcore_axis_name="s")

    @functools.partial(
        pl.kernel, mesh=mesh,
        out_type=jax.ShapeDtypeStruct((B_TOT, E_DIM), jnp.float32),
        scratch_types=[
            pltpu.VMEM((CH,), jnp.int32),
            pltpu.VMEM((CH, E_DIM), jnp.float32),
            pltpu.SemaphoreType.DMA,
        ],
    )
    def gk(table_hbm, idx_hbm, out_hbm, idx_v, rows_v, sem):
        wid = lax.axis_index("s") * NC + lax.axis_index("c")
        base = wid * per_w
        for j in range(n_ch):
            off = base + j * CH
            pltpu.sync_copy(idx_hbm.at[pl.ds(off, CH)], idx_v)
            pltpu.async_copy(table_hbm.at[idx_v], rows_v, sem).wait()
            pltpu.sync_copy(rows_v, out_hbm.at[pl.ds(off, CH)])

    return gk(embedding, idx)


def kernel(z, embedding):
    z3 = z.reshape(NB, E_DIM, M_PER_B)
    zf2d = jnp.transpose(z, (0, 2, 3, 1)).reshape(B_TOT, E_DIM)
    z2rows = _z2_rows(zf2d).reshape(NB, 1, M_PER_B)
    e_pad = jnp.concatenate(
        [embedding, jnp.full((N_PAD - N_E, E_DIM), 1e18, jnp.float32)], axis=0)
    idx_flat, loss = _dist_argmin(z3, z2rows, e_pad)
    zq_flat = _sc_gather(embedding, idx_flat)
    z_q = zq_flat.reshape(NB, 32, 32, E_DIM).transpose(0, 3, 1, 2)
    return z_q, loss[0, 0], idx_flat


# speedup vs baseline: 1.9183x; 1.0327x over previous
import functools

import jax
import jax.numpy as jnp
from jax import lax
from jax.experimental import pallas as pl
from jax.experimental.pallas import tpu as pltpu
from jax.experimental.pallas import tpu_sc as plsc

N_E = 8192
E_DIM = 256
NB = 16
M_PER_B = 1024
B_TOT = NB * M_PER_B
LOSS_SCALE = 1.25 / (B_TOT * E_DIM)

N_BLK = 2736
N_STEPS = 3


def _z2_body(zf_ref, o_ref):
    zfb = zf_ref[...]
    o_ref[...] = jnp.sum(zfb * zfb, axis=1)


def _z2_rows(zf2d):
    return pl.pallas_call(
        _z2_body,
        grid=(NB,),
        in_specs=[pl.BlockSpec((M_PER_B, E_DIM), lambda b: (b, 0))],
        out_specs=pl.BlockSpec((M_PER_B,), lambda b: (b,)),
        out_shape=jax.ShapeDtypeStruct((B_TOT,), jnp.float32),
    )(zf2d)


def _dist_argmin_body(z_ref, z2_ref, e_ref, idx_ref, loss_ref, selv_ref,
                      seli_ref, carry_ref):
    b = pl.program_id(0)
    n = pl.program_id(1)
    n0 = n * N_BLK

    zb = z_ref[0]
    eb = e_ref[...]
    mm = lax.dot_general(eb, zb, (((1,), (0,)), ((), ())),
                         preferred_element_type=jnp.float32,
                         precision=lax.Precision.DEFAULT)
    z2 = z2_ref[0, 0]
    e2 = jnp.sum(eb * eb, axis=1)
    d = (z2[None, :] + e2[:, None]) - 2.0 * mm
    gidx = lax.broadcasted_iota(jnp.int32, d.shape, 0) + n0

    def _winmin(dm):
        v = jnp.min(dm, axis=0)
        i = jnp.min(jnp.where(dm == v[None, :], gidx, jnp.int32(N_E)),
                    axis=0)
        return v, i

    @pl.when(n == 0)
    def _():
        vw, iw = _winmin(d)
        selv_ref[...] = vw
        seli_ref[...] = iw
        carry_ref[...] = vw.astype(jnp.bfloat16).astype(jnp.float32)

    @pl.when(n == 1)
    def _():
        vw, iw = _winmin(d)
        win = vw < carry_ref[...]
        selv_ref[...] = jnp.where(win, vw, selv_ref[...])
        seli_ref[...] = jnp.where(win, iw, seli_ref[...])
        carry_ref[...] = jnp.where(
            win, vw.astype(jnp.bfloat16).astype(jnp.float32), carry_ref[...])

    @pl.when(n == N_STEPS - 1)
    def _():
        vw, iw = _winmin(jnp.where(gidx < N_E, d, jnp.inf))
        win = vw < carry_ref[...]
        idx_ref[...] = jnp.where(win, iw, seli_ref[...])
        s = jnp.sum(jnp.where(win, vw, selv_ref[...]))
        prev = jnp.where(b == 0, jnp.float32(0.0), loss_ref[0, 0])
        tot = prev + s
        loss_ref[0, 0] = jnp.where(b == NB - 1, tot * LOSS_SCALE, tot)


def _dist_argmin(z3, z2rows, embedding, interpret=False):
    return pl.pallas_call(
        _dist_argmin_body,
        grid=(NB, N_STEPS),
        in_specs=[
            pl.BlockSpec((1, E_DIM, M_PER_B), lambda b, n: (b, 0, 0)),
            pl.BlockSpec((1, 1, M_PER_B), lambda b, n: (b, 0, 0)),
            pl.BlockSpec((N_BLK, E_DIM), lambda b, n: (n, 0)),
        ],
        out_specs=[
            pl.BlockSpec((M_PER_B,), lambda b, n: (b,)),
            pl.BlockSpec((1, 1), lambda b, n: (0, 0),
                         memory_space=pltpu.SMEM),
        ],
        out_shape=[
            jax.ShapeDtypeStruct((B_TOT,), jnp.int32),
            jax.ShapeDtypeStruct((1, 1), jnp.float32),
        ],
        scratch_shapes=[
            pltpu.VMEM((M_PER_B,), jnp.float32),
            pltpu.VMEM((M_PER_B,), jnp.int32),
            pltpu.VMEM((M_PER_B,), jnp.float32),
        ],
        interpret=interpret,
    )(z3, z2rows, embedding)


def _sc_gather(embedding, idx):
    NC, NS = 2, 16
    NW = NC * NS
    CH = 128
    per_w = B_TOT // NW
    n_ch = per_w // CH
    mesh = plsc.VectorSubcoreMesh(core_axis_name="c", subcore_axis_name="s")

    @functools.partial(
        pl.kernel, mesh=mesh,
        out_type=jax.ShapeDtypeStruct((B_TOT, E_DIM), jnp.float32),
        scratch_types=[
            pltpu.VMEM((CH,), jnp.int32),
            pltpu.VMEM((CH, E_DIM), jnp.float32),
            pltpu.SemaphoreType.DMA,
        ],
    )
    def gk(table_hbm, idx_hbm, out_hbm, idx_v, rows_v, sem):
        wid = lax.axis_index("s") * NC + lax.axis_index("c")
        base = wid * per_w
        for j in range(n_ch):
            off = base + j * CH
            pltpu.sync_copy(idx_hbm.at[pl.ds(off, CH)], idx_v)
            pltpu.async_copy(table_hbm.at[idx_v], rows_v, sem).wait()
            pltpu.sync_copy(rows_v, out_hbm.at[pl.ds(off, CH)])

    return gk(embedding, idx)


def kernel(z, embedding):
    z3 = z.reshape(NB, E_DIM, M_PER_B)
    zf2d = jnp.transpose(z, (0, 2, 3, 1)).reshape(B_TOT, E_DIM)
    z2rows = _z2_rows(zf2d).reshape(NB, 1, M_PER_B)
    idx_flat, loss = _dist_argmin(z3, z2rows, embedding)
    zq_flat = _sc_gather(embedding, idx_flat)
    z_q = zq_flat.reshape(NB, 32, 32, E_DIM).transpose(0, 3, 1, 2)
    return z_q, loss[0, 0], idx_flat
